# Initial kernel scaffold; baseline (speedup 1.0000x reference)
import jax
import jax.numpy as jnp
from jax.experimental import pallas as pl


def kernel(global_features, my_features, my_lidar, enemy_features, last_known_enemy_features, my_positions, enemy_positions, last_known_enemy_positions, enemy_mask, W1, b1, W2, b2, W3, b3, ln_scale, ln_bias, train):
    raise NotImplementedError("write your pallas kernel here")


# speedup vs baseline: 2.3196x; 2.3196x over previous
import jax
import jax.numpy as jnp
from jax import lax
from jax.experimental import pallas as pl

_RES = 16
_BS = 32


def _leaky(x):
    return jnp.where(x >= 0, x, 0.01 * x)


def _conv_s2(x, Wr, b, H, C, K, bs):
    Ho = H // 2
    xr = x.reshape(bs, Ho, 2, H, C)
    even, odd = xr[:, :, 0], xr[:, :, 1]
    rows = {
        0: even,
        1: odd,
        2: jnp.concatenate(
            [even[:, 1:], jnp.zeros_like(even[:, :1])], axis=1),
    }
    acc = jnp.zeros((bs * Ho * Ho, K), jnp.float32)
    for ky in (0, 1, 2):
        y = rows[ky]
        yc = y.reshape(bs, Ho, Ho, 2, C)
        ceven, codd = yc[:, :, :, 0], yc[:, :, :, 1]
        cols = {
            0: ceven,
            1: codd,
            2: jnp.concatenate(
                [ceven[:, :, 1:], jnp.zeros_like(ceven[:, :, :1])], axis=2),
        }
        for kx in (0, 1, 2):
            t = ky * 3 + kx
            tap = cols[kx].reshape(bs * Ho * Ho, C)
            acc = acc + jnp.dot(tap, Wr[t],
                                preferred_element_type=jnp.float32)
    return (acc + b).reshape(bs, Ho, Ho, K)


def _body(g_ref, myf_ref, enf_ref, lkf_ref,
          myx_ref, myy_ref, enx_ref, eny_ref, lkx_ref, lky_ref,
          emask_ref, lidar_ref,
          w1_ref, b1_ref, w2_ref, b2_ref, w3_ref, b3_ref,
          lns_ref, lnb_ref,
          team_ref, agent_ref):
    bs = _BS
    g = g_ref[...]
    myf = myf_ref[...]
    enf = enf_ref[...]
    lkf = lkf_ref[...]
    emask = emask_ref[...]

    feats = (myf, enf, lkf)
    xs = (myx_ref[...], enx_ref[...], lkx_ref[...])
    ys = (myy_ref[...], eny_ref[...], lky_ref[...])
    masks = (jnp.ones_like(emask), emask, jnp.ones_like(emask))

    iota_p = lax.broadcasted_iota(jnp.int32, (1, _RES * _RES), 1)

    delta = jnp.zeros((bs, _RES * _RES, 64), jnp.float32)
    cnt = jnp.zeros((bs, _RES * _RES), jnp.float32)
    for s in range(3):
        f = feats[s]
        ix = jnp.clip((xs[s] * _RES).astype(jnp.int32), 0, _RES - 1)
        iy = jnp.clip((ys[s] * _RES).astype(jnp.int32), 0, _RES - 1)
        c = iy * _RES + ix
        valid = jnp.logical_and(f[:, :, 0] == 1.0, masks[s] == 1.0)
        vf = valid.astype(jnp.float32)
        for j in range(8):
            m = (iota_p == c[:, j:j + 1]).astype(jnp.float32) \
                * vf[:, j:j + 1]
            delta = delta + m[:, :, None] * f[:, j, :][:, None, :]
            cnt = cnt + m

    inv = 1.0 / (1.0 + cnt)
    mm = (g[:, None, :] + delta) * inv[:, :, None]
    x = mm.reshape(bs, _RES, _RES, 64)

    o = _leaky(_conv_s2(x, w1_ref[...], b1_ref[...], 16, 64, 32, bs))
    o = _leaky(_conv_s2(o, w2_ref[...], b2_ref[...], 8, 32, 64, bs))
    o = _conv_s2(o, w3_ref[...], b3_ref[...], 4, 64, 64, bs)

    flat = jnp.concatenate(
        [o[:, 0, 0, :], o[:, 0, 1, :], o[:, 1, 0, :], o[:, 1, 1, :]],
        axis=-1)
    mu = jnp.mean(flat, axis=-1, keepdims=True)
    var = jnp.mean(jnp.square(flat - mu), axis=-1, keepdims=True)
    y = (flat - mu) / jnp.sqrt(var + 1e-6) * lns_ref[...] + lnb_ref[...]
    team_ref[...] = _leaky(y)

    agent_ref[...] = jnp.concatenate([myf, lidar_ref[...]], axis=-1)


def kernel(global_features, my_features, my_lidar, enemy_features,
           last_known_enemy_features, my_positions, enemy_positions,
           last_known_enemy_positions, enemy_mask,
           W1, b1, W2, b2, W3, b3, ln_scale, ln_bias, train):
    B, T, D = my_features.shape
    L = my_lidar.shape[-1]
    bs = _BS

    w1r = W1.reshape(9, D, 32)
    w2r = W2.reshape(9, 32, 64)
    w3r = W3.reshape(9, 64, 64)

    def bspec(shape3, idx):
        return pl.BlockSpec(shape3, idx)

    grid = (B // bs,)
    bmap = lambda i: (i, 0)
    bmap3 = lambda i: (i, 0, 0)
    wmap2 = lambda i: (0, 0)
    wmap3 = lambda i: (0, 0, 0)

    in_specs = [
        bspec((bs, D), bmap),
        bspec((bs, T, D), bmap3),
        bspec((bs, T, D), bmap3),
        bspec((bs, T, D), bmap3),
        bspec((bs, T), bmap),
        bspec((bs, T), bmap),
        bspec((bs, T), bmap),
        bspec((bs, T), bmap),
        bspec((bs, T), bmap),
        bspec((bs, T), bmap),
        bspec((bs, T), bmap),
        bspec((bs, T, L), bmap3),
        bspec((9, D, 32), wmap3),
        bspec((1, 32), wmap2),
        bspec((9, 32, 64), wmap3),
        bspec((1, 64), wmap2),
        bspec((9, 64, 64), wmap3),
        bspec((1, 64), wmap2),
        bspec((1, 256), wmap2),
        bspec((1, 256), wmap2),
    ]
    out_specs = [
        bspec((bs, 256), bmap),
        bspec((bs, T, D + L), bmap3),
    ]
    out_shape = [
        jax.ShapeDtypeStruct((B, 256), jnp.float32),
        jax.ShapeDtypeStruct((B, T, D + L), jnp.float32),
    ]

    team, agent = pl.pallas_call(
        _body,
        grid=grid,
        in_specs=in_specs,
        out_specs=out_specs,
        out_shape=out_shape,
    )(
        global_features, my_features, enemy_features,
        last_known_enemy_features,
        my_positions[..., 0], my_positions[..., 1],
        enemy_positions[..., 0], enemy_positions[..., 1],
        last_known_enemy_positions[..., 0],
        last_known_enemy_positions[..., 1],
        enemy_mask, my_lidar,
        w1r, b1.reshape(1, 32), w2r, b2.reshape(1, 64),
        w3r, b3.reshape(1, 64),
        ln_scale.reshape(1, 256), ln_bias.reshape(1, 256),
    )
    return (team, agent)


# speedup vs baseline: 9.9895x; 4.3066x over previous
import jax
import jax.numpy as jnp
from jax import lax
from jax.experimental import pallas as pl

_RES = 16
_BS = 32


def _leaky(x):
    return jnp.where(x >= 0, x, 0.01 * x)


def _conv_s2(x, Wc, b, H, C, K, bs):
    Ho = H // 2
    xr = x.reshape(bs, Ho, 2, H, C)
    even, odd = xr[:, :, 0], xr[:, :, 1]
    rows = {
        0: even,
        1: odd,
        2: jnp.concatenate(
            [even[:, 1:], jnp.zeros_like(even[:, :1])], axis=1),
    }
    taps = []
    for ky in (0, 1, 2):
        yc = rows[ky].reshape(bs, Ho, Ho, 2, C)
        ceven, codd = yc[:, :, :, 0], yc[:, :, :, 1]
        cols = {
            0: ceven,
            1: codd,
            2: jnp.concatenate(
                [ceven[:, :, 1:], jnp.zeros_like(ceven[:, :, :1])], axis=2),
        }
        for kx in (0, 1, 2):
            taps.append(cols[kx])
    big = jnp.concatenate(taps, axis=-1).reshape(bs * Ho * Ho, 9 * C)
    acc = jnp.dot(big, Wc, preferred_element_type=jnp.float32)
    return (acc + b).reshape(bs, Ho, Ho, K)


def _body(g_ref, myf_ref, enf_ref, lkf_ref,
          myx_ref, myy_ref, enx_ref, eny_ref, lkx_ref, lky_ref,
          emask_ref, lidar_ref,
          w1_ref, b1_ref, w2_ref, b2_ref, w3_ref, b3_ref,
          lns_ref, lnb_ref,
          team_ref, agent_ref):
    bs = _BS
    g = g_ref[...]
    myf = myf_ref[...]
    enf = enf_ref[...]
    lkf = lkf_ref[...]
    emask = emask_ref[...]

    feats = (myf, enf, lkf)
    xs = (myx_ref[...], enx_ref[...], lkx_ref[...])
    ys = (myy_ref[...], eny_ref[...], lky_ref[...])
    masks = (jnp.ones_like(emask), emask, jnp.ones_like(emask))

    cs = []
    obsxs = []
    for s in range(3):
        f = feats[s]
        ix = jnp.clip((xs[s] * _RES).astype(jnp.int32), 0, _RES - 1)
        iy = jnp.clip((ys[s] * _RES).astype(jnp.int32), 0, _RES - 1)
        cs.append(iy * _RES + ix)
        valid = jnp.logical_and(f[:, :, 0] == 1.0, masks[s] == 1.0)
        vf = valid.astype(jnp.float32)
        obsxs.append(jnp.concatenate(
            [f * vf[:, :, None], vf[:, :, None]], axis=-1))
    c = jnp.concatenate(cs, axis=1)
    obsx = jnp.concatenate(obsxs, axis=1)

    iota_p = lax.broadcasted_iota(jnp.int32, (bs, _RES * _RES, 24), 1)
    oh = (iota_p == c[:, None, :]).astype(jnp.float32)
    acc = lax.dot_general(oh, obsx, (((2,), (1,)), ((0,), (0,))),
                          preferred_element_type=jnp.float32)

    inv = 1.0 / (1.0 + acc[:, :, 64])
    mm = (g[:, None, :] + acc[:, :, :64]) * inv[:, :, None]
    x = mm.reshape(bs, _RES, _RES, 64)

    o = _leaky(_conv_s2(x, w1_ref[...], b1_ref[...], 16, 64, 32, bs))
    o = _leaky(_conv_s2(o, w2_ref[...], b2_ref[...], 8, 32, 64, bs))
    o = _conv_s2(o, w3_ref[...], b3_ref[...], 4, 64, 64, bs)

    flat = jnp.concatenate(
        [o[:, 0, 0, :], o[:, 0, 1, :], o[:, 1, 0, :], o[:, 1, 1, :]],
        axis=-1)
    mu = jnp.mean(flat, axis=-1, keepdims=True)
    var = jnp.mean(jnp.square(flat - mu), axis=-1, keepdims=True)
    y = (flat - mu) / jnp.sqrt(var + 1e-6) * lns_ref[...] + lnb_ref[...]
    team_ref[...] = _leaky(y)

    agent_ref[...] = jnp.concatenate([myf, lidar_ref[...]], axis=-1)


def kernel(global_features, my_features, my_lidar, enemy_features,
           last_known_enemy_features, my_positions, enemy_positions,
           last_known_enemy_positions, enemy_mask,
           W1, b1, W2, b2, W3, b3, ln_scale, ln_bias, train):
    B, T, D = my_features.shape
    L = my_lidar.shape[-1]
    bs = _BS

    w1r = W1.reshape(9 * D, 32)
    w2r = W2.reshape(9 * 32, 64)
    w3r = W3.reshape(9 * 64, 64)

    def bspec(shape3, idx):
        return pl.BlockSpec(shape3, idx)

    grid = (B // bs,)
    bmap = lambda i: (i, 0)
    bmap3 = lambda i: (i, 0, 0)
    wmap2 = lambda i: (0, 0)
    wmap3 = lambda i: (0, 0, 0)

    in_specs = [
        bspec((bs, D), bmap),
        bspec((bs, T, D), bmap3),
        bspec((bs, T, D), bmap3),
        bspec((bs, T, D), bmap3),
        bspec((bs, T), bmap),
        bspec((bs, T), bmap),
        bspec((bs, T), bmap),
        bspec((bs, T), bmap),
        bspec((bs, T), bmap),
        bspec((bs, T), bmap),
        bspec((bs, T), bmap),
        bspec((bs, T, L), bmap3),
        bspec((9 * D, 32), wmap2),
        bspec((1, 32), wmap2),
        bspec((9 * 32, 64), wmap2),
        bspec((1, 64), wmap2),
        bspec((9 * 64, 64), wmap2),
        bspec((1, 64), wmap2),
        bspec((1, 256), wmap2),
        bspec((1, 256), wmap2),
    ]
    out_specs = [
        bspec((bs, 256), bmap),
        bspec((bs, T, D + L), bmap3),
    ]
    out_shape = [
        jax.ShapeDtypeStruct((B, 256), jnp.float32),
        jax.ShapeDtypeStruct((B, T, D + L), jnp.float32),
    ]

    team, agent = pl.pallas_call(
        _body,
        grid=grid,
        in_specs=in_specs,
        out_specs=out_specs,
        out_shape=out_shape,
    )(
        global_features, my_features, enemy_features,
        last_known_enemy_features,
        my_positions[..., 0], my_positions[..., 1],
        enemy_positions[..., 0], enemy_positions[..., 1],
        last_known_enemy_positions[..., 0],
        last_known_enemy_positions[..., 1],
        enemy_mask, my_lidar,
        w1r, b1.reshape(1, 32), w2r, b2.reshape(1, 64),
        w3r, b3.reshape(1, 64),
        ln_scale.reshape(1, 256), ln_bias.reshape(1, 256),
    )
    return (team, agent)
